# baseline (speedup 1.0000x reference)
import functools

import jax
import jax.numpy as jnp
from jax import lax
from jax.experimental import pallas as pl
from jax.experimental.pallas import tpu as pltpu
from jax.experimental.pallas import tpu_sc as plsc

BATCH = 16384
EMBED_DIM = 32
LANES = 16

_info = plsc.get_sparse_core_info()
NC, NS = _info.num_cores, _info.num_subcores
NW = NC * NS
BPW = BATCH // NW
GROUPS = BPW // LANES


def _mf_body(uids, iids, eu, ei, bu, bi, gb, out,
             uid_v, iid_v, urows, irows, bu_v, bi_v, gb_v, out_v,
             sem_u, sem_i, sem_bu, sem_bi):
    wid = lax.axis_index("s") * NC + lax.axis_index("c")
    base = wid * BPW

    pltpu.sync_copy(uids.at[pl.ds(base, BPW)], uid_v)
    pltpu.sync_copy(iids.at[pl.ds(base, BPW)], iid_v)
    pltpu.sync_copy(gb, gb_v.at[pl.ds(0, 1)])

    cu = pltpu.async_copy(eu.at[uid_v], urows, sem_u)
    ci = pltpu.async_copy(ei.at[iid_v], irows, sem_i)
    cbu = pltpu.async_copy(bu.at[uid_v], bu_v, sem_bu)
    cbi = pltpu.async_copy(bi.at[iid_v], bi_v, sem_bi)
    cu.wait()
    ci.wait()
    cbu.wait()
    cbi.wait()

    lanes = lax.iota(jnp.int32, LANES)
    gbs = gb_v[...][0]

    def group(g, carry):
        row = g * LANES + lanes
        acc = jnp.zeros((LANES,), jnp.float32)
        for k in range(EMBED_DIM):
            col = (lanes + k) & (EMBED_DIM - 1)
            acc = acc + (plsc.load_gather(urows, [row, col])
                         * plsc.load_gather(irows, [row, col]))
        o = g * LANES
        out_v[pl.ds(o, LANES)] = (acc + bu_v[pl.ds(o, LANES)]
                                  + bi_v[pl.ds(o, LANES)] + gbs)
        return carry

    lax.fori_loop(0, GROUPS, group, 0)
    pltpu.sync_copy(out_v, out.at[pl.ds(base, BPW)])


@jax.jit
def _mf(user_ids, item_ids, embedding_users, embedding_items,
        bias_users, bias_items, global_bias):
    mesh = plsc.VectorSubcoreMesh(core_axis_name="c", subcore_axis_name="s")
    run = pl.kernel(
        _mf_body,
        mesh=mesh,
        out_type=jax.ShapeDtypeStruct((BATCH,), jnp.float32),
        compiler_params=pltpu.CompilerParams(
            needs_layout_passes=False, use_tc_tiling_on_sc=False),
        scratch_types=[
            pltpu.VMEM((BPW,), jnp.int32),
            pltpu.VMEM((BPW,), jnp.int32),
            pltpu.VMEM((BPW, EMBED_DIM), jnp.float32),
            pltpu.VMEM((BPW, EMBED_DIM), jnp.float32),
            pltpu.VMEM((BPW,), jnp.float32),
            pltpu.VMEM((BPW,), jnp.float32),
            pltpu.VMEM((LANES,), jnp.float32),
            pltpu.VMEM((BPW,), jnp.float32),
            pltpu.SemaphoreType.DMA,
            pltpu.SemaphoreType.DMA,
            pltpu.SemaphoreType.DMA,
            pltpu.SemaphoreType.DMA,
        ],
    )
    return run(user_ids, item_ids, embedding_users, embedding_items,
               bias_users, bias_items, global_bias)


def kernel(user_ids, item_ids, embedding_users, embedding_items,
           bias_users, bias_items, global_bias):
    return _mf(user_ids, item_ids, embedding_users, embedding_items,
               bias_users, bias_items, global_bias)
